# baseline (speedup 1.0000x reference)
import functools

import jax
import jax.numpy as jnp
from jax import lax
from jax.experimental import pallas as pl
from jax.experimental.pallas import tpu as pltpu
from jax.experimental.pallas import tpu_sc as plsc

_B, _H, _W, _C = 4, 112, 112, 96
_HO, _WO = 224, 224
_F = _HO * _WO * _C
_E = _B * _H * _W * _C
_NW = 32
_PER_W = _E // _NW
_K = 10752
_NCHUNK = _PER_W // _K

_mesh = plsc.VectorSubcoreMesh(core_axis_name="c", subcore_axis_name="s")


@functools.partial(
    pl.kernel,
    out_type=(),
    mesh=_mesh,
    scratch_types=[
        pltpu.VMEM((_K,), jnp.int32),
        pltpu.VMEM((_K,), jnp.int32),
        pltpu.VMEM((_K,), jnp.float32),
        pltpu.SemaphoreType.DMA,
    ],
)
def _scatter_ones(idx_hbm, sw_ref, idx_v, dst_v, ones_v, sem):
    wid = lax.axis_index("s") * 2 + lax.axis_index("c")
    b = wid >> 3
    base = wid * _PER_W
    dst_off = b * _F
    one = jnp.full((16,), 1.0, jnp.float32)

    def fill_body(j, carry):
        ones_v[pl.ds(pl.multiple_of(j * 16, 16), 16)] = one
        return carry

    lax.fori_loop(0, _K // 16, fill_body, 0, unroll=8)

    def chunk_body(k, carry):
        pltpu.sync_copy(idx_hbm.at[pl.ds(base + k * _K, _K)], idx_v)

        def vec_body(j, carry2):
            sl = pl.ds(pl.multiple_of(j * 16, 16), 16)
            dst_v[sl] = idx_v[sl] + dst_off
            return carry2

        lax.fori_loop(0, _K // 16, vec_body, 0, unroll=8)
        pltpu.async_copy(ones_v, sw_ref.at[dst_v], sem).wait()
        return carry

    lax.fori_loop(0, _NCHUNK, chunk_body, 0)


def _mul_body(pooled_ref, sw_ref, out_ref):
    p = pooled_ref[...].reshape(_HB, _W, _C)
    two = jnp.concatenate([p, p], axis=-1)
    s = sw_ref[...].reshape(_HB, _W, 2 * _C)
    out_ref[...] = (two * s).reshape(1, _HB, 1, _W, 2 * _C)


_HB = 16


def _mask_mul(pooled, sw5):
    return pl.pallas_call(
        _mul_body,
        grid=(_B, _H // _HB, 2),
        in_specs=[
            pl.BlockSpec((1, _HB, _W, _C), lambda b, h, dh: (b, h, 0, 0)),
            pl.BlockSpec((1, _HB, 1, _W, 2 * _C),
                         lambda b, h, dh: (b, h, dh, 0, 0)),
        ],
        out_specs=pl.BlockSpec((1, _HB, 1, _W, 2 * _C),
                               lambda b, h, dh: (b, h, dh, 0, 0)),
        out_shape=jax.ShapeDtypeStruct((_B, _H, 2, _W, 2 * _C), jnp.float32),
    )(pooled, sw5)


def kernel(pooled_Maps, indices, Rectified_FM):
    del Rectified_FM
    idx_flat = indices.reshape(-1)
    sw_ref = jax.new_ref(jnp.zeros((_B * _F,), jnp.float32))
    _scatter_ones(idx_flat, sw_ref)
    sw5 = sw_ref[...].reshape(_B, _H, 2, _W, 2 * _C)
    out5 = _mask_mul(pooled_Maps, sw5)
    return out5.reshape(_B, _HO, _WO, _C)


# speedup vs baseline: 1.1139x; 1.1139x over previous
import functools

import jax
import jax.numpy as jnp
import numpy as np
from jax import lax
from jax.experimental import pallas as pl
from jax.experimental.pallas import tpu as pltpu
from jax.experimental.pallas import tpu_sc as plsc

_B, _H, _W, _C = 4, 112, 112, 96
_HO, _WO = 224, 224
_F = _HO * _WO * _C
_N = _H * _W * _C
_E = _B * _N
_PER_T = _E // 16
_K = 10752
_NCHUNK = _PER_T // _K

_NR = 2
_WSPB = _B * _N // (2 * _NR)
_WSL = _WSPB // 16
_ZB = _WSL // 8

_RECIP96 = np.float32(1.0 / 96.0)
_RECIP224 = np.float32((1.0 + 2.0**-21) / 224.0)

_mesh = plsc.VectorSubcoreMesh(core_axis_name="c", subcore_axis_name="s")


@functools.partial(
    pl.kernel,
    out_type=jax.ShapeDtypeStruct((_B * _N,), jnp.int32),
    mesh=_mesh,
    scratch_types=[
        pltpu.VMEM_SHARED((_WSPB + 16,), jnp.int32),
        pltpu.VMEM((_K,), jnp.int32),
        pltpu.VMEM((_K,), jnp.int32),
        pltpu.VMEM((_K,), jnp.int32),
        pltpu.VMEM((_ZB,), jnp.int32),
        pltpu.SemaphoreType.DMA,
    ],
)
def _scatter_rounds(idx_hbm, sw, spm, idx_v, loc_v, val_v, zb, sem):
    sc = lax.axis_index("c")
    sid = lax.axis_index("s")
    word_off = (sid >> 2) * _N
    base = sid * _PER_T
    zeros16 = jnp.zeros((16,), jnp.int32)

    def fill_zero(j, c):
        zb[pl.ds(pl.multiple_of(j * 16, 16), 16)] = zeros16
        return c

    lax.fori_loop(0, _ZB // 16, fill_zero, 0, unroll=8)

    for r in range(_NR):
        lo = (2 * r + sc) * _WSPB
        for z in range(_WSL // _ZB):
            pltpu.sync_copy(zb, spm.at[pl.ds(sid * _WSL + z * _ZB, _ZB)])
        plsc.subcore_barrier()

        def chunk(k, c, lo=lo):
            pltpu.sync_copy(idx_hbm.at[pl.ds(base + k * _K, _K)], idx_v)

            def grp(j, c2, lo=lo):
                sl = pl.ds(pl.multiple_of(j * 16, 16), 16)
                i = idx_v[sl]
                q = (i.astype(jnp.float32) * _RECIP96).astype(jnp.int32)
                ch = i - q * 96
                ho = (q.astype(jnp.float32) * _RECIP224).astype(jnp.int32)
                wo = q - ho * 224
                word = (lax.shift_right_logical(ho, 1) * (_W * _C)
                        + lax.shift_right_logical(wo, 1) * _C
                        + ch) + word_off
                local = word - lo
                hit = (local >= 0) & (local < _WSPB)
                loc_v[sl] = jnp.where(hit, local, _WSPB)
                field = ((ho & 1) << 1) | (wo & 1)
                val_v[sl] = lax.shift_left(1, lax.shift_left(field, 3))
                return c2

            lax.fori_loop(0, _K // 16, grp, 0, unroll=4)
            pltpu.async_copy(val_v, spm.at[loc_v], sem, add=True).wait()
            return c

        lax.fori_loop(0, _NCHUNK, chunk, 0)
        plsc.subcore_barrier()
        pltpu.sync_copy(
            spm.at[pl.ds(sid * _WSL, _WSL)],
            sw.at[pl.ds(lo + sid * _WSL, _WSL)],
        )
        plsc.subcore_barrier()


def _mul_body(pooled_ref, sw_ref, out_ref):
    dh = pl.program_id(2)
    p = pooled_ref[...].reshape(_HB, _W, _C)
    s = sw_ref[...].reshape(_HB, _W, _C)
    shift = dh * 16
    fields = lax.shift_right_logical(s, shift)
    m0 = ((fields & 0xFF) != 0).astype(jnp.float32)
    m1 = ((fields & 0xFF00) != 0).astype(jnp.float32)
    out = jnp.concatenate([p * m0, p * m1], axis=-1)
    out_ref[...] = out.reshape(1, _HB, 1, _W, 2 * _C)


_HB = 16


def _mask_mul(pooled, sw4):
    return pl.pallas_call(
        _mul_body,
        grid=(_B, _H // _HB, 2),
        in_specs=[
            pl.BlockSpec((1, _HB, _W, _C), lambda b, h, dh: (b, h, 0, 0)),
            pl.BlockSpec((1, _HB, _W, _C), lambda b, h, dh: (b, h, 0, 0)),
        ],
        out_specs=pl.BlockSpec((1, _HB, 1, _W, 2 * _C),
                               lambda b, h, dh: (b, h, dh, 0, 0)),
        out_shape=jax.ShapeDtypeStruct((_B, _H, 2, _W, 2 * _C), jnp.float32),
    )(pooled, sw4)


def kernel(pooled_Maps, indices, Rectified_FM):
    del Rectified_FM
    idx_flat = indices.reshape(-1)
    sw_words = _scatter_rounds(idx_flat)
    sw4 = sw_words.reshape(_B, _H, _W, _C)
    out5 = _mask_mul(pooled_Maps, sw4)
    return out5.reshape(_B, _HO, _WO, _C)


# speedup vs baseline: 3.7486x; 3.3653x over previous
import functools

import jax
import jax.numpy as jnp
import numpy as np
from jax import lax
from jax.experimental import pallas as pl
from jax.experimental.pallas import tpu as pltpu
from jax.experimental.pallas import tpu_sc as plsc

_B, _H, _W, _C = 4, 112, 112, 96
_HO, _WO = 224, 224
_F = _HO * _WO * _C
_N = _H * _W * _C
_E = _B * _N
_PER_T = _E // 16
_K = 10752
_NCHUNK = _PER_T // _K

_NR = 2
_WSPB = _B * _N // (2 * _NR)
_WSL = _WSPB // 16
_ZB = _WSL // 8
_DUMP = 8192

_RECIP96 = np.float32(1.0 / 96.0)
_RECIP224 = np.float32((1.0 + 2.0**-21) / 224.0)

_mesh = plsc.VectorSubcoreMesh(core_axis_name="c", subcore_axis_name="s")


@functools.partial(
    pl.kernel,
    out_type=jax.ShapeDtypeStruct((_B * _N,), jnp.int32),
    mesh=_mesh,
    scratch_types=[
        pltpu.VMEM_SHARED((_WSPB + _DUMP,), jnp.int32),
        pltpu.VMEM((_K,), jnp.int32),
        pltpu.VMEM((_K,), jnp.int32),
        pltpu.VMEM((_K,), jnp.int32),
        pltpu.VMEM((_ZB,), jnp.int32),
        pltpu.SemaphoreType.DMA,
    ],
)
def _scatter_rounds(idx_hbm, sw, spm, idx_v, loc_v, val_v, zb, sem):
    sc = lax.axis_index("c")
    sid = lax.axis_index("s")
    word_off = (sid >> 2) * _N
    base = sid * _PER_T
    zeros16 = jnp.zeros((16,), jnp.int32)

    def fill_zero(j, c):
        zb[pl.ds(pl.multiple_of(j * 16, 16), 16)] = zeros16
        return c

    lax.fori_loop(0, _ZB // 16, fill_zero, 0, unroll=8)

    for r in range(_NR):
        lo = (2 * r + sc) * _WSPB
        for z in range(_WSL // _ZB):
            pltpu.sync_copy(zb, spm.at[pl.ds(sid * _WSL + z * _ZB, _ZB)])
        plsc.subcore_barrier()

        def chunk(k, c, lo=lo):
            pltpu.sync_copy(idx_hbm.at[pl.ds(base + k * _K, _K)], idx_v)

            def grp(j, c2, lo=lo):
                sl = pl.ds(pl.multiple_of(j * 16, 16), 16)
                i = idx_v[sl]
                q = (i.astype(jnp.float32) * _RECIP96).astype(jnp.int32)
                ch = i - q * 96
                ho = (q.astype(jnp.float32) * _RECIP224).astype(jnp.int32)
                wo = q - ho * 224
                word = (lax.shift_right_logical(ho, 1) * (_W * _C)
                        + lax.shift_right_logical(wo, 1) * _C
                        + ch) + word_off
                local = word - lo
                hit = (local >= 0) & (local < _WSPB)
                loc_v[sl] = jnp.where(hit, local, _WSPB + (i & (_DUMP - 1)))
                field = ((ho & 1) << 1) | (wo & 1)
                val_v[sl] = lax.shift_left(1, lax.shift_left(field, 3))
                return c2

            lax.fori_loop(0, _K // 16, grp, 0, unroll=4)
            pltpu.async_copy(val_v, spm.at[loc_v], sem, add=True).wait()
            return c

        lax.fori_loop(0, _NCHUNK, chunk, 0)
        plsc.subcore_barrier()
        pltpu.sync_copy(
            spm.at[pl.ds(sid * _WSL, _WSL)],
            sw.at[pl.ds(lo + sid * _WSL, _WSL)],
        )
        plsc.subcore_barrier()


def _mul_body(pooled_ref, sw_ref, out_ref):
    dh = pl.program_id(2)
    p = pooled_ref[...].reshape(_HB, _W, _C)
    s = sw_ref[...].reshape(_HB, _W, _C)
    shift = dh * 16
    fields = lax.shift_right_logical(s, shift)
    m0 = ((fields & 0xFF) != 0).astype(jnp.float32)
    m1 = ((fields & 0xFF00) != 0).astype(jnp.float32)
    out = jnp.concatenate([p * m0, p * m1], axis=-1)
    out_ref[...] = out.reshape(1, _HB, 1, _W, 2 * _C)


_HB = 16


def _mask_mul(pooled, sw4):
    return pl.pallas_call(
        _mul_body,
        grid=(_B, _H // _HB, 2),
        in_specs=[
            pl.BlockSpec((1, _HB, _W, _C), lambda b, h, dh: (b, h, 0, 0)),
            pl.BlockSpec((1, _HB, _W, _C), lambda b, h, dh: (b, h, 0, 0)),
        ],
        out_specs=pl.BlockSpec((1, _HB, 1, _W, 2 * _C),
                               lambda b, h, dh: (b, h, dh, 0, 0)),
        out_shape=jax.ShapeDtypeStruct((_B, _H, 2, _W, 2 * _C), jnp.float32),
    )(pooled, sw4)


def kernel(pooled_Maps, indices, Rectified_FM):
    del Rectified_FM
    idx_flat = indices.reshape(-1)
    sw_words = _scatter_rounds(idx_flat)
    sw4 = sw_words.reshape(_B, _H, _W, _C)
    out5 = _mask_mul(pooled_Maps, sw4)
    return out5.reshape(_B, _HO, _WO, _C)


# speedup vs baseline: 6.0121x; 1.6038x over previous
import functools

import jax
import jax.numpy as jnp
import numpy as np
from jax import lax
from jax.experimental import pallas as pl
from jax.experimental.pallas import tpu as pltpu
from jax.experimental.pallas import tpu_sc as plsc

_B, _H, _W, _C = 4, 112, 112, 96
_HO, _WO = 224, 224
_F = _HO * _WO * _C
_N = _H * _W * _C
_E = _B * _N
_PER_T = _E // 16
_K = 10752
_NCHUNK = _PER_T // _K

_NWORD = _B * (_H // 2) * _W * _C
_WSPB = _NWORD // 2
_WSL = _WSPB // 16
_ZB = _WSL // 8
_DUMP = 8192

_RECIP96 = np.float32(1.0 / 96.0)
_RECIP224 = np.float32((1.0 + 2.0**-21) / 224.0)

_mesh = plsc.VectorSubcoreMesh(core_axis_name="c", subcore_axis_name="s")


@functools.partial(
    pl.kernel,
    out_type=jax.ShapeDtypeStruct((_NWORD,), jnp.int32),
    mesh=_mesh,
    scratch_types=[
        pltpu.VMEM_SHARED((_WSPB + _DUMP,), jnp.int32),
        pltpu.VMEM((_K,), jnp.int32),
        pltpu.VMEM((_K,), jnp.int32),
        pltpu.VMEM((_K,), jnp.int32),
        pltpu.VMEM((_ZB,), jnp.int32),
        pltpu.SemaphoreType.DMA,
    ],
)
def _scatter_rounds(idx_hbm, sw, spm, idx_v, loc_v, val_v, zb, sem):
    sc = lax.axis_index("c")
    sid = lax.axis_index("s")
    word_off = (sid >> 2) * (_NWORD // _B)
    base = sid * _PER_T
    lo = sc * _WSPB
    zeros16 = jnp.zeros((16,), jnp.int32)

    def fill_zero(j, c):
        zb[pl.ds(pl.multiple_of(j * 16, 16), 16)] = zeros16
        return c

    lax.fori_loop(0, _ZB // 16, fill_zero, 0, unroll=8)

    for z in range(_WSL // _ZB):
        pltpu.sync_copy(zb, spm.at[pl.ds(sid * _WSL + z * _ZB, _ZB)])
    plsc.subcore_barrier()

    def chunk(k, c):
        pltpu.sync_copy(idx_hbm.at[pl.ds(base + k * _K, _K)], idx_v)

        def grp(j, c2):
            sl = pl.ds(pl.multiple_of(j * 16, 16), 16)
            i = idx_v[sl]
            q = (i.astype(jnp.float32) * _RECIP96).astype(jnp.int32)
            ch = i - q * 96
            ho = (q.astype(jnp.float32) * _RECIP224).astype(jnp.int32)
            wo = q - ho * 224
            word = (lax.shift_right_logical(ho, 2) * (_W * _C)
                    + lax.shift_right_logical(wo, 1) * _C
                    + ch) + word_off
            local = word - lo
            hit = (local >= 0) & (local < _WSPB)
            loc_v[sl] = jnp.where(hit, local, _WSPB + (i & (_DUMP - 1)))
            val = (jnp.where((wo & 1) != 0, 16, 1)
                   * jnp.where((ho & 2) != 0, 256, 1)
                   * jnp.where((ho & 1) != 0, 65536, 1))
            val_v[sl] = val
            return c2

        lax.fori_loop(0, _K // 16, grp, 0, unroll=4)
        pltpu.async_copy(val_v, spm.at[loc_v], sem, add=True).wait()
        return c

    lax.fori_loop(0, _NCHUNK, chunk, 0)
    plsc.subcore_barrier()
    pltpu.sync_copy(
        spm.at[pl.ds(sid * _WSL, _WSL)],
        sw.at[pl.ds(lo + sid * _WSL, _WSL)],
    )


def _mul_body(pooled_ref, sw_ref, out_ref):
    dh = pl.program_id(2)
    p = pooled_ref[...].reshape(_HB, _W, _C)
    s = sw_ref[...].reshape(_HB // 2, _W, _C)
    srep = jnp.repeat(s, 2, axis=0)
    hp = lax.broadcasted_iota(jnp.int32, (_HB, 1, 1), 0) & 1
    fields = lax.shift_right_logical(srep, dh * 16 + hp * 8)
    m0 = ((fields & 0xF) != 0).astype(jnp.float32)
    m1 = ((fields & 0xF0) != 0).astype(jnp.float32)
    out = jnp.concatenate([p * m0, p * m1], axis=-1)
    out_ref[...] = out.reshape(1, _HB, 1, _W, 2 * _C)


_HB = 16


def _mask_mul(pooled, sw4):
    return pl.pallas_call(
        _mul_body,
        grid=(_B, _H // _HB, 2),
        in_specs=[
            pl.BlockSpec((1, _HB, _W, _C), lambda b, h, dh: (b, h, 0, 0)),
            pl.BlockSpec((1, _HB // 2, _W, _C), lambda b, h, dh: (b, h, 0, 0)),
        ],
        out_specs=pl.BlockSpec((1, _HB, 1, _W, 2 * _C),
                               lambda b, h, dh: (b, h, dh, 0, 0)),
        out_shape=jax.ShapeDtypeStruct((_B, _H, 2, _W, 2 * _C), jnp.float32),
    )(pooled, sw4)


def kernel(pooled_Maps, indices, Rectified_FM):
    del Rectified_FM
    idx_flat = indices.reshape(-1)
    sw_words = _scatter_rounds(idx_flat)
    sw4 = sw_words.reshape(_B, _H // 2, _W, _C)
    out5 = _mask_mul(pooled_Maps, sw4)
    return out5.reshape(_B, _HO, _WO, _C)


# speedup vs baseline: 6.5392x; 1.0877x over previous
import functools

import jax
import jax.numpy as jnp
import numpy as np
from jax import lax
from jax.experimental import pallas as pl
from jax.experimental.pallas import tpu as pltpu
from jax.experimental.pallas import tpu_sc as plsc

_B, _H, _W, _C = 4, 112, 112, 96
_HO, _WO = 224, 224
_F = _HO * _WO * _C
_N = _H * _W * _C
_E = _B * _N
_PER_T = _E // 16
_K = 5376
_NCHUNK = _PER_T // _K

_NWORD = _B * (_H // 2) * _W * _C
_WSPB = _NWORD // 2
_WSL = _WSPB // 16
_ZB = _WSL // 8
_DUMP = 8192

_RECIP96 = np.float32(1.0 / 96.0)
_RECIP224 = np.float32((1.0 + 2.0**-21) / 224.0)

_mesh = plsc.VectorSubcoreMesh(core_axis_name="c", subcore_axis_name="s")


@functools.partial(
    pl.kernel,
    out_type=jax.ShapeDtypeStruct((_NWORD,), jnp.int32),
    mesh=_mesh,
    scratch_types=[
        pltpu.VMEM_SHARED((_WSPB + _DUMP,), jnp.int32),
        pltpu.VMEM((_K,), jnp.int32),
        pltpu.VMEM((_K,), jnp.int32),
        pltpu.VMEM((_K,), jnp.int32),
        pltpu.VMEM((_K,), jnp.int32),
        pltpu.VMEM((_K,), jnp.int32),
        pltpu.VMEM((_K,), jnp.int32),
        pltpu.VMEM((_ZB,), jnp.int32),
        pltpu.SemaphoreType.DMA,
        pltpu.SemaphoreType.DMA,
    ],
)
def _scatter_rounds(idx_hbm, sw, spm, idx_v0, idx_v1, loc_v0, loc_v1, val_v0,
                    val_v1, zb, sem0, sem1):
    idx_b, loc_b, val_b = (idx_v0, idx_v1), (loc_v0, loc_v1), (val_v0, val_v1)
    sems = (sem0, sem1)
    sc = lax.axis_index("c")
    sid = lax.axis_index("s")
    word_off = (sid >> 2) * (_NWORD // _B)
    base = sid * _PER_T
    lo = sc * _WSPB
    zeros16 = jnp.zeros((16,), jnp.int32)

    def fill_zero(j, c):
        zb[pl.ds(pl.multiple_of(j * 16, 16), 16)] = zeros16
        return c

    lax.fori_loop(0, _ZB // 16, fill_zero, 0, unroll=8)

    for z in range(_WSL // _ZB):
        pltpu.sync_copy(zb, spm.at[pl.ds(sid * _WSL + z * _ZB, _ZB)])
    plsc.subcore_barrier()

    def do_chunk(k, s):
        idx_v, loc_v, val_v = idx_b[s], loc_b[s], val_b[s]
        pltpu.sync_copy(idx_hbm.at[pl.ds(base + k * _K, _K)], idx_v)

        def grp(j, c2):
            sl = pl.ds(pl.multiple_of(j * 16, 16), 16)
            i = idx_v[sl]
            q = (i.astype(jnp.float32) * _RECIP96).astype(jnp.int32)
            ch = i - q * 96
            ho = (q.astype(jnp.float32) * _RECIP224).astype(jnp.int32)
            wo = q - ho * 224
            word = (lax.shift_right_logical(ho, 2) * (_W * _C)
                    + lax.shift_right_logical(wo, 1) * _C
                    + ch) + word_off
            local = word - lo
            hit = (local >= 0) & (local < _WSPB)
            loc_v[sl] = jnp.where(hit, local, _WSPB + (i & (_DUMP - 1)))
            val = (jnp.where((wo & 1) != 0, 16, 1)
                   * jnp.where((ho & 2) != 0, 256, 1)
                   * jnp.where((ho & 1) != 0, 65536, 1))
            val_v[sl] = val
            return c2

        lax.fori_loop(0, _K // 16, grp, 0, unroll=4)
        pltpu.async_copy(val_v, spm.at[loc_v], sems[s], add=True)

    def drain(s):
        pltpu.make_async_copy(val_b[s], spm.at[loc_b[s]], sems[s]).wait()

    def pair(k2, c):
        for s in range(2):
            @pl.when(k2 > 0)
            def _(s=s):
                drain(s)
            do_chunk(k2 * 2 + s, s)
        return c

    lax.fori_loop(0, _NCHUNK // 2, pair, 0)
    drain(0)
    drain(1)
    plsc.subcore_barrier()
    pltpu.sync_copy(
        spm.at[pl.ds(sid * _WSL, _WSL)],
        sw.at[pl.ds(lo + sid * _WSL, _WSL)],
    )


def _mul_body(pooled_ref, sw_ref, out_ref):
    dh = pl.program_id(2)
    p = pooled_ref[...].reshape(_HB, _W, _C)
    s = sw_ref[...].reshape(_HB // 2, _W, _C)
    srep = jnp.repeat(s, 2, axis=0)
    hp = lax.broadcasted_iota(jnp.int32, (_HB, 1, 1), 0) & 1
    fields = lax.shift_right_logical(srep, dh * 16 + hp * 8)
    m0 = ((fields & 0xF) != 0).astype(jnp.float32)
    m1 = ((fields & 0xF0) != 0).astype(jnp.float32)
    out = jnp.concatenate([p * m0, p * m1], axis=-1)
    out_ref[...] = out.reshape(1, _HB, 1, _W, 2 * _C)


_HB = 16


def _mask_mul(pooled, sw4):
    return pl.pallas_call(
        _mul_body,
        grid=(_B, _H // _HB, 2),
        in_specs=[
            pl.BlockSpec((1, _HB, _W, _C), lambda b, h, dh: (b, h, 0, 0)),
            pl.BlockSpec((1, _HB // 2, _W, _C), lambda b, h, dh: (b, h, 0, 0)),
        ],
        out_specs=pl.BlockSpec((1, _HB, 1, _W, 2 * _C),
                               lambda b, h, dh: (b, h, dh, 0, 0)),
        out_shape=jax.ShapeDtypeStruct((_B, _H, 2, _W, 2 * _C), jnp.float32),
    )(pooled, sw4)


def kernel(pooled_Maps, indices, Rectified_FM):
    del Rectified_FM
    idx_flat = indices.reshape(-1)
    sw_words = _scatter_rounds(idx_flat)
    sw4 = sw_words.reshape(_B, _H // 2, _W, _C)
    out5 = _mask_mul(pooled_Maps, sw4)
    return out5.reshape(_B, _HO, _WO, _C)
